# baseline (speedup 1.0000x reference)
import functools

import jax
import jax.numpy as jnp
from jax import lax
from jax.experimental import pallas as pl
from jax.experimental.pallas import tpu as pltpu
from jax.experimental.pallas import tpu_sc as plsc

_D = 32
_NC = 2
_NS = 16
_NW = _NC * _NS
_BW = 512


@jax.jit
def _sc_embedding_gather(table, idx_t):
    s_len, b_len = idx_t.shape
    n_units = (s_len * b_len) // _BW
    u_per_w = n_units // _NW
    assert u_per_w % 2 == 0
    n_bq = b_len // _BW
    mesh = plsc.VectorSubcoreMesh(core_axis_name="c", subcore_axis_name="s")

    @functools.partial(
        pl.kernel,
        out_type=jax.ShapeDtypeStruct((s_len, _D, b_len), jnp.float32),
        mesh=mesh,
        scratch_types=(
            [pltpu.VMEM((_BW,), jnp.int32) for _ in range(2)]
            + [pltpu.VMEM((_BW, _D), jnp.float32) for _ in range(2)]
            + [
                pltpu.VMEM((_D, _BW), jnp.float32),
                pltpu.SemaphoreType.DMA,
                pltpu.SemaphoreType.DMA,
            ]
        ),
        compiler_params=pltpu.CompilerParams(
            use_tc_tiling_on_sc=False, needs_layout_passes=False
        ),
    )
    def k(table_hbm, idx_hbm, out_hbm, i0, i1, r0, r1, plane_v, g0, g1):
        idx_v = (i0, i1)
        rows_v = (r0, r1)
        gsem = (g0, g1)
        wid = lax.axis_index("s") * _NC + lax.axis_index("c")

        def unit_coords(u):
            g = u * _NW + wid
            return g // n_bq, (g % n_bq) * _BW

        def start_gather(u, slot):
            s, b0 = unit_coords(u)
            pltpu.sync_copy(idx_hbm.at[s, pl.ds(b0, _BW)], idx_v[slot])
            pltpu.async_copy(table_hbm.at[idx_v[slot]], rows_v[slot], gsem[slot])

        def wait_gather(slot):
            pltpu.make_async_copy(
                table_hbm.at[idx_v[slot]], rows_v[slot], gsem[slot]
            ).wait()

        iota = lax.iota(jnp.int32, 16)
        dsplats = [jnp.full((16,), d, dtype=jnp.int32) for d in range(_D)]

        def transpose_unit(slot):
            rv = rows_v[slot]

            @plsc.parallel_loop(0, _BW, step=16, unroll=4)
            def _(g0):
                gvec = iota + g0
                for d in range(_D):
                    v = plsc.load_gather(rv, [gvec, dsplats[d]])
                    plane_v[d, pl.ds(g0, 16)] = v

        def store_plane(u):
            s, b0 = unit_coords(u)
            pltpu.sync_copy(plane_v, out_hbm.at[s, :, pl.ds(b0, _BW)])

        start_gather(0, 0)

        def body(p, carry):
            for c in range(2):
                u = p * 2 + c
                nxt = 1 - c

                @pl.when(u + 1 < u_per_w)
                def _():
                    start_gather(u + 1, nxt)

                wait_gather(c)
                transpose_unit(c)
                store_plane(u)
            return carry

        lax.fori_loop(0, u_per_w // 2, body, 0)

    return k(table, idx_t)


def kernel(inputs, table):
    b, s = inputs.shape
    idx_t = inputs.T
    out3 = _sc_embedding_gather(table, idx_t)
    return jnp.transpose(out3, (2, 0, 1))


# speedup vs baseline: 1.0458x; 1.0458x over previous
import functools

import jax
import jax.numpy as jnp
from jax import lax
from jax.experimental import pallas as pl
from jax.experimental.pallas import tpu as pltpu
from jax.experimental.pallas import tpu_sc as plsc

_D = 32
_NC = 2
_NS = 16
_NW = _NC * _NS
_BW = 512


@jax.jit
def _sc_embedding_gather(table, idx_t):
    s_len, b_len = idx_t.shape
    n_units = (s_len * b_len) // _BW
    u_per_w = n_units // _NW
    assert u_per_w % 2 == 0
    n_bq = b_len // _BW
    mesh = plsc.VectorSubcoreMesh(core_axis_name="c", subcore_axis_name="s")

    @functools.partial(
        pl.kernel,
        out_type=jax.ShapeDtypeStruct((s_len, _D, b_len), jnp.float32),
        mesh=mesh,
        scratch_types=(
            [pltpu.VMEM((_BW,), jnp.int32) for _ in range(2)]
            + [pltpu.VMEM((_BW, _D), jnp.float32) for _ in range(2)]
            + [
                pltpu.VMEM((_D * _BW,), jnp.float32),
                pltpu.SemaphoreType.DMA,
                pltpu.SemaphoreType.DMA,
                pltpu.SemaphoreType.DMA,
            ]
        ),
        compiler_params=pltpu.CompilerParams(
            use_tc_tiling_on_sc=False, needs_layout_passes=False
        ),
    )
    def k(table_hbm, idx_hbm, out_hbm, i0, i1, r0, r1, plane_v, g0, g1, psem):
        idx_v = (i0, i1)
        rows_v = (r0, r1)
        gsem = (g0, g1)
        wid = lax.axis_index("s") * _NC + lax.axis_index("c")

        def unit_coords(u):
            g = u * _NW + wid
            return g // n_bq, (g % n_bq) * _BW

        def start_gather(u, slot):
            s, b0 = unit_coords(u)
            pltpu.sync_copy(idx_hbm.at[s, pl.ds(b0, _BW)], idx_v[slot])
            pltpu.async_copy(table_hbm.at[idx_v[slot]], rows_v[slot], gsem[slot])

        def wait_gather(slot):
            pltpu.make_async_copy(
                table_hbm.at[idx_v[slot]], rows_v[slot], gsem[slot]
            ).wait()

        iota = lax.iota(jnp.int32, 16)
        c_lo = iota * _BW
        c_hi = (iota + 16) * _BW

        def transpose_unit(slot):
            rv = rows_v[slot]

            @plsc.parallel_loop(0, _BW, step=1, unroll=8)
            def _(g):
                gv = jnp.full((16,), g, dtype=jnp.int32)
                plsc.store_scatter(plane_v, [c_lo + gv], rv[g, pl.ds(0, 16)])
                plsc.store_scatter(plane_v, [c_hi + gv], rv[g, pl.ds(16, 16)])

        def store_plane(u):
            s, b0 = unit_coords(u)
            for d in range(_D):
                pltpu.async_copy(
                    plane_v.at[pl.ds(d * _BW, _BW)],
                    out_hbm.at[s, d, pl.ds(b0, _BW)],
                    psem,
                )
            for d in range(_D):
                pltpu.make_async_copy(
                    plane_v.at[pl.ds(d * _BW, _BW)],
                    out_hbm.at[s, d, pl.ds(b0, _BW)],
                    psem,
                ).wait()

        start_gather(0, 0)

        def body(p, carry):
            for c in range(2):
                u = p * 2 + c
                nxt = 1 - c

                @pl.when(u + 1 < u_per_w)
                def _():
                    start_gather(u + 1, nxt)

                wait_gather(c)
                transpose_unit(c)
                store_plane(u)
            return carry

        lax.fori_loop(0, u_per_w // 2, body, 0)

    return k(table, idx_t)


def kernel(inputs, table):
    b, s = inputs.shape
    idx_t = inputs.T
    out3 = _sc_embedding_gather(table, idx_t)
    return jnp.transpose(out3, (2, 0, 1))


# speedup vs baseline: 1.0914x; 1.0435x over previous
import functools

import jax
import jax.numpy as jnp
from jax import lax
from jax.experimental import pallas as pl
from jax.experimental.pallas import tpu as pltpu
from jax.experimental.pallas import tpu_sc as plsc

_D = 32
_NC = 2
_NS = 16
_NW = _NC * _NS
_NBUF = 4


@functools.partial(jax.jit, static_argnames=("chunk",))
def _sc_embedding_gather(table, idx_flat, *, chunk):
    b = idx_flat.shape[0]
    b_per_w = b // _NW
    n_chunks = b_per_w // chunk
    assert n_chunks % _NBUF == 0 and n_chunks >= _NBUF
    mesh = plsc.VectorSubcoreMesh(core_axis_name="c", subcore_axis_name="s")

    @functools.partial(
        pl.kernel,
        out_type=jax.ShapeDtypeStruct((b, _D), jnp.float32),
        mesh=mesh,
        scratch_types=(
            [pltpu.VMEM((chunk,), jnp.int32) for _ in range(_NBUF)]
            + [pltpu.VMEM((chunk, _D), jnp.float32) for _ in range(_NBUF)]
            + [pltpu.SemaphoreType.DMA for _ in range(2 * _NBUF)]
        ),
        compiler_params=pltpu.CompilerParams(use_tc_tiling_on_sc=False),
    )
    def k(table_hbm, idx_hbm, out_hbm, *scratch):
        idx_v = scratch[:_NBUF]
        rows_v = scratch[_NBUF : 2 * _NBUF]
        gsem = scratch[2 * _NBUF : 3 * _NBUF]
        ssem = scratch[3 * _NBUF : 4 * _NBUF]
        wid = lax.axis_index("s") * _NC + lax.axis_index("c")
        base = wid * b_per_w

        def start_gather(ci, slot):
            off = base + ci * chunk
            pltpu.sync_copy(idx_hbm.at[pl.ds(off, chunk)], idx_v[slot])
            pltpu.async_copy(
                table_hbm.at[idx_v[slot]], rows_v[slot], gsem[slot]
            )

        def wait_gather(slot):
            pltpu.make_async_copy(
                table_hbm.at[idx_v[slot]], rows_v[slot], gsem[slot]
            ).wait()

        def start_store(ci, slot):
            off = base + ci * chunk
            pltpu.async_copy(
                rows_v[slot], out_hbm.at[pl.ds(off, chunk)], ssem[slot]
            )

        def wait_store(ci, slot):
            off = base + ci * chunk
            pltpu.make_async_copy(
                rows_v[slot], out_hbm.at[pl.ds(off, chunk)], ssem[slot]
            ).wait()

        start_gather(0, 0)
        start_gather(1, 1)

        def body(p, carry):
            for s in range(_NBUF):
                i = p * _NBUF + s
                wait_gather(s)
                start_store(i, s)
                nxt = (s + 2) % _NBUF

                @pl.when(jnp.logical_and(i + 2 < n_chunks, i >= 2))
                def _():
                    wait_store(i - 2, nxt)

                @pl.when(i + 2 < n_chunks)
                def _():
                    start_gather(i + 2, nxt)
            return carry

        lax.fori_loop(0, n_chunks // _NBUF, body, 0)
        for j in range(n_chunks - _NBUF, n_chunks):
            wait_store(j, j % _NBUF)

    return k(table, idx_flat)


def kernel(inputs, table):
    b, s = inputs.shape
    idx_flat = inputs.reshape(b * s).astype(jnp.int32)
    out = _sc_embedding_gather(table, idx_flat, chunk=800)
    return out.reshape(b, s, _D)
